# baseline (speedup 1.0000x reference)
import functools

import jax
import jax.numpy as jnp
from jax import lax
from jax.experimental import pallas as pl
from jax.experimental.pallas import tpu as pltpu
from jax.experimental.pallas import tpu_sc as plsc

B = 1024
T = 20
DEPTH = 1000

NUM_CORES = 2
NUM_SUBCORES = 16
NW = NUM_CORES * NUM_SUBCORES

DT = DEPTH // 8
BT = B // 128
UPW = 4
NBUF = 2


def _sc_one_hot(x_hbm, zsrc_hbm, out_hbm, x_v, zbuf, *sems):
    wid = lax.axis_index("s") * NUM_CORES + lax.axis_index("c")
    dt0 = jnp.minimum(wid * UPW, DT - UPW)
    d_lo = dt0 * 8

    zfills = [
        pltpu.async_copy(zsrc_hbm, zbuf.at[nb], sems[nb]) for nb in range(NBUF)
    ]
    pltpu.sync_copy(x_hbm, x_v)

    iota = lax.iota(jnp.int32, 16)
    iota_t = iota * T
    one_f = jnp.full((16,), 1.0, jnp.float32)
    zero_f = jnp.zeros((16,), jnp.float32)

    def scan(t, nb, val):
        nbv = jnp.full((16,), nb, jnp.int32)

        def body(v, carry):
            b0 = v * 16
            xv = plsc.load_gather(x_v, [iota_t + (b0 * T + t)])
            u = xv - d_lo
            m = plsc.bitcast(u, jnp.uint32) < jnp.uint32(UPW * 8)
            unit = lax.shift_right_arithmetic(u, 3)
            di = lax.bitwise_and(u, 7)
            btv = jnp.broadcast_to(lax.shift_right_logical(v, 3), (16,))
            bi = iota + lax.bitwise_and(b0, 127)
            plsc.store_scatter(zbuf, [nbv, unit, btv, di, bi], val, mask=m)
            return carry

        lax.fori_loop(0, B // 16, body, 0)

    handles = {}
    for j in range(T):
        nb = j % NBUF
        if j < NBUF:
            zfills[nb].wait()
        if j >= NBUF:
            handles.pop(j - NBUF).wait()
            scan(j - NBUF, nb, zero_f)
        scan(j, nb, one_f)
        handles[j] = pltpu.async_copy(
            zbuf.at[nb], out_hbm.at[j, pl.ds(dt0, UPW)], sems[nb]
        )
    for j in sorted(handles):
        handles[j].wait()


@jax.jit
def _one_hot(x):
    zsrc = jnp.zeros((UPW, BT, 8, 128), jnp.float32)
    run = functools.partial(
        pl.kernel,
        out_type=jax.ShapeDtypeStruct((T, DT, BT, 8, 128), jnp.float32),
        mesh=plsc.VectorSubcoreMesh(core_axis_name="c", subcore_axis_name="s"),
        scratch_types=[
            pltpu.VMEM((B * T,), jnp.int32),
            pltpu.VMEM((NBUF, UPW, BT, 8, 128), jnp.float32),
        ] + [pltpu.SemaphoreType.DMA] * NBUF,
        compiler_params=pltpu.CompilerParams(needs_layout_passes=False),
    )(_sc_one_hot)
    out5 = run(x.reshape(B * T), zsrc)
    return out5.transpose(2, 4, 1, 3, 0).reshape(B, DEPTH, T)


def kernel(X_in, ones):
    del ones
    return _one_hot(X_in.astype(jnp.int32))


# speedup vs baseline: 1.0135x; 1.0135x over previous
import functools

import jax
import jax.numpy as jnp
from jax import lax
from jax.experimental import pallas as pl
from jax.experimental.pallas import tpu as pltpu
from jax.experimental.pallas import tpu_sc as plsc

B = 1024
T = 20
DEPTH = 1000

NUM_CORES = 2
NUM_SUBCORES = 16
NW = NUM_CORES * NUM_SUBCORES

DT = DEPTH // 8
BT = B // 128
UPW = 4
NBUF = 2


def _sc_one_hot(x_hbm, zsrc_hbm, out_hbm, x_v, zbuf, *sems):
    wid = lax.axis_index("s") * NUM_CORES + lax.axis_index("c")
    dt0 = jnp.minimum(wid * UPW, DT - UPW)
    d_lo = dt0 * 8

    zfills = [
        pltpu.async_copy(zsrc_hbm, zbuf.at[nb], sems[nb]) for nb in range(NBUF)
    ]
    pltpu.sync_copy(x_hbm, x_v)
    for h in zfills:
        h.wait()

    iota = lax.iota(jnp.int32, 16)
    iota_t = iota * T
    one_f = jnp.full((16,), 1.0, jnp.float32)
    zero_f = jnp.zeros((16,), jnp.float32)

    def scan(t, nb, val):
        nbv = jnp.full((16,), nb, jnp.int32)

        def body(v, carry):
            b0 = v * 16
            xv = plsc.load_gather(x_v, [iota_t + (b0 * T + t)])
            u = xv - d_lo
            m = plsc.bitcast(u, jnp.uint32) < jnp.uint32(UPW * 8)
            unit = lax.shift_right_arithmetic(u, 3)
            di = lax.bitwise_and(u, 7)
            btv = jnp.broadcast_to(lax.shift_right_logical(v, 3), (16,))
            bi = iota + lax.bitwise_and(b0, 127)
            plsc.store_scatter(zbuf, [nbv, unit, btv, di, bi], val, mask=m)
            return carry

        lax.fori_loop(0, B // 16, body, 0)

    handles = {}
    for j in range(T):
        nb = j % NBUF
        if j >= NBUF:
            handles.pop(j - NBUF).wait()
            scan(j - NBUF, nb, zero_f)
        scan(j, nb, one_f)
        handles[j] = pltpu.async_copy(
            zbuf.at[nb], out_hbm.at[j, pl.ds(dt0, UPW)], sems[nb]
        )
    for j in sorted(handles):
        handles[j].wait()


@jax.jit
def _one_hot(x):
    zsrc = jnp.zeros((UPW, BT, 8, 128), jnp.float32)
    run = functools.partial(
        pl.kernel,
        out_type=jax.ShapeDtypeStruct((T, DT, BT, 8, 128), jnp.float32),
        mesh=plsc.VectorSubcoreMesh(core_axis_name="c", subcore_axis_name="s"),
        scratch_types=[
            pltpu.VMEM((B * T,), jnp.int32),
            pltpu.VMEM((NBUF, UPW, BT, 8, 128), jnp.float32),
        ] + [pltpu.SemaphoreType.DMA] * NBUF,
        compiler_params=pltpu.CompilerParams(needs_layout_passes=False),
    )(_sc_one_hot)
    out5 = run(x.reshape(B * T), zsrc)
    return out5.transpose(2, 4, 1, 3, 0).reshape(B, DEPTH, T)


def kernel(X_in, ones):
    del ones
    return _one_hot(X_in.astype(jnp.int32))


# speedup vs baseline: 1.0180x; 1.0044x over previous
import functools

import jax
import jax.numpy as jnp
import numpy as np
from jax import lax
from jax.experimental import pallas as pl
from jax.experimental.pallas import tpu as pltpu
from jax.experimental.pallas import tpu_sc as plsc

B = 1024
T = 20
DEPTH = 1000

NUM_CORES = 2
NUM_SUBCORES = 16
NW = NUM_CORES * NUM_SUBCORES

DT = DEPTH // 8
BT = B // 128
UPW = 4
NBUF = 2


def _sc_one_hot(x_hbm, zsrc_hbm, out_hbm, x_v, zbuf, *sems):
    wid = lax.axis_index("s") * NUM_CORES + lax.axis_index("c")
    dt0 = jnp.minimum(wid * UPW, DT - UPW)
    d_lo = dt0 * 8

    zfills = [
        pltpu.async_copy(zsrc_hbm, zbuf.at[nb], sems[nb]) for nb in range(NBUF)
    ]
    pltpu.sync_copy(x_hbm, x_v)
    for h in zfills:
        h.wait()

    iota = lax.iota(jnp.int32, 16)
    iota_t = iota * T
    one_f = jnp.full((16,), 1.0, jnp.float32)
    zero_f = jnp.zeros((16,), jnp.float32)

    def scan(t, nb, val):
        nbv = jnp.full((16,), nb, jnp.int32)

        def body(v, carry):
            b0 = v * 16
            xv = plsc.load_gather(x_v, [iota_t + (b0 * T + t)])
            u = xv - d_lo
            m = plsc.bitcast(u, jnp.uint32) < jnp.uint32(UPW * 8)
            unit = lax.shift_right_arithmetic(u, 3)
            di = lax.bitwise_and(u, 7)
            btv = jnp.broadcast_to(lax.shift_right_logical(v, 3), (16,))
            bi = iota + lax.bitwise_and(b0, 127)
            plsc.store_scatter(zbuf, [nbv, unit, btv, di, bi], val, mask=m)
            return carry

        lax.fori_loop(0, B // 16, body, 0)

    handles = {}
    for j in range(T):
        nb = j % NBUF
        if j >= NBUF:
            handles.pop(j - NBUF).wait()
            scan(j - NBUF, nb, zero_f)
        scan(j, nb, one_f)
        handles[j] = pltpu.async_copy(
            zbuf.at[nb], out_hbm.at[j, pl.ds(dt0, UPW)], sems[nb]
        )
    for j in sorted(handles):
        handles[j].wait()


_ZSRC = np.zeros((UPW, BT, 8, 128), np.float32)


@jax.jit
def _one_hot(x):
    zsrc = jnp.asarray(_ZSRC)
    run = functools.partial(
        pl.kernel,
        out_type=jax.ShapeDtypeStruct((T, DT, BT, 8, 128), jnp.float32),
        mesh=plsc.VectorSubcoreMesh(core_axis_name="c", subcore_axis_name="s"),
        scratch_types=[
            pltpu.VMEM((B * T,), jnp.int32),
            pltpu.VMEM((NBUF, UPW, BT, 8, 128), jnp.float32),
        ] + [pltpu.SemaphoreType.DMA] * NBUF,
        compiler_params=pltpu.CompilerParams(needs_layout_passes=False),
    )(_sc_one_hot)
    out5 = run(x.reshape(B * T), zsrc)
    return out5.transpose(2, 4, 1, 3, 0).reshape(B, DEPTH, T)


def kernel(X_in, ones):
    del ones
    return _one_hot(X_in.astype(jnp.int32))
